# Initial kernel scaffold; baseline (speedup 1.0000x reference)
import jax
import jax.numpy as jnp
from jax.experimental import pallas as pl


def kernel(x_fea, graph_pool_avg, padded_nei, adj, candidate, h_g_m_pooled, mask_operation, gin0_w1, gin0_b1, gin0_w2, gin0_b2, gin1_w1, gin1_b1, gin1_w2, gin1_b2, gin2_w1, gin2_b1, gin2_w2, gin2_b2, a_w1, a_b1, a_w2, a_b2, a_w3, a_b3, c_w1, c_b1, c_w2, c_b2, c_w3, c_b3):
    raise NotImplementedError("write your pallas kernel here")


# speedup vs baseline: 1.0436x; 1.0436x over previous
import jax
import jax.numpy as jnp
from jax.experimental import pallas as pl

_HI = jax.lax.Precision.HIGHEST


def _bdot(a, b):
    return jnp.dot(a.astype(jnp.bfloat16), b.astype(jnp.bfloat16),
                   preferred_element_type=jnp.float32)


def _fused_step(
    x_ref, adj_ref, cand_ref, mask_ref, hgm_ref, pool_ref,
    g0w1_ref, g0b1_ref, g0w2_ref, g0b2_ref,
    g1w1_ref, g1b1_ref, g1w2_ref, g1b2_ref,
    g2w1_ref, g2b1_ref, g2w2_ref, g2b2_ref,
    aw1a_ref, aw1b_ref, aw1c_ref, ab1_ref,
    aw2_ref, ab2_ref, aw3_ref, ab3_ref,
    cw1_ref, cb1_ref, cw2_ref, cb2_ref, cw3_ref, cb3_ref,
    ti_ref, ai_ref, la_ref, pr_ref, hg_ref, jv_ref,
):
    f32 = jnp.float32
    a16 = adj_ref[0]
    h = x_ref[0]
    T = a16.shape[0]

    gin = ((g0w1_ref, g0b1_ref, g0w2_ref, g0b2_ref),
           (g1w1_ref, g1b1_ref, g1w2_ref, g1b2_ref),
           (g2w1_ref, g2b1_ref, g2w2_ref, g2b2_ref))
    for w1_ref, b1_ref, w2_ref, b2_ref in gin:
        neigh = jnp.dot(a16, h.astype(jnp.bfloat16),
                        preferred_element_type=f32)
        pooled = neigh + h
        m = jnp.maximum(_bdot(pooled, w1_ref[...]) + b1_ref[...], 0.0)
        m = _bdot(m, w2_ref[...]) + b2_ref[...]
        h = jnp.maximum(m, 0.0)

    h16f = h.astype(jnp.bfloat16).astype(f32)
    hg = jnp.dot(pool_ref[...], h16f, preferred_element_type=f32,
                 precision=_HI)

    cand = cand_ref[0]
    nj = cand.shape[0]
    iota_t = jax.lax.broadcasted_iota(jnp.int32, (nj, T), 1)
    onehot = (iota_t == cand).astype(f32)
    cf = jnp.dot(onehot, h, preferred_element_type=f32, precision=_HI)

    hgm = hgm_ref[0]
    row = _bdot(hg, aw1b_ref[...]) + _bdot(hgm, aw1c_ref[...]) + ab1_ref[...]
    t1 = jnp.tanh(_bdot(cf, aw1a_ref[...]) + row)
    t2 = jnp.tanh(_bdot(t1, aw2_ref[...]) + ab2_ref[...])
    sc = _bdot(t2, aw3_ref[...]) + ab3_ref[...]
    sc = sc - mask_ref[0] * 1e30

    smax = jnp.max(sc, axis=0, keepdims=True)
    e = jnp.exp(sc - smax)
    esum = jnp.sum(e, axis=0, keepdims=True)
    prob = e / esum

    iota_nj = jax.lax.broadcasted_iota(jnp.int32, (nj, 1), 0)
    pmax = jnp.max(prob, axis=0, keepdims=True)
    am = jnp.min(jnp.where(prob == pmax, iota_nj, nj), axis=0, keepdims=True)
    task = jnp.sum(jnp.where(iota_nj == am, cand, 0), axis=0, keepdims=True)
    la = jnp.log(pmax + 1e-10)

    v1 = jnp.tanh(_bdot(hg, cw1_ref[...]) + cb1_ref[...])
    v2 = jnp.tanh(_bdot(v1, cw2_ref[...]) + cb2_ref[...])
    jv = _bdot(v2, cw3_ref[...]) + cb3_ref[...]

    ti_ref[...] = task.reshape(1, 1, 1)
    ai_ref[...] = am.reshape(1, 1, 1)
    la_ref[...] = la.reshape(1, 1, 1)
    pr_ref[...] = prob.reshape(1, nj, 1)
    hg_ref[...] = hg.reshape(1, 1, -1)
    jv_ref[...] = jv.reshape(1, 1, -1)


def kernel(x_fea, graph_pool_avg, padded_nei, adj, candidate, h_g_m_pooled,
           mask_operation,
           gin0_w1, gin0_b1, gin0_w2, gin0_b2,
           gin1_w1, gin1_b1, gin1_w2, gin1_b2,
           gin2_w1, gin2_b1, gin2_w2, gin2_b2,
           a_w1, a_b1, a_w2, a_b2, a_w3, a_b3,
           c_w1, c_b1, c_w2, c_b2, c_w3, c_b3):
    B, T, _ = adj.shape
    NJ = candidate.shape[1]
    DIN = x_fea.shape[1]
    H = gin0_w2.shape[0]
    f32 = jnp.float32

    x3 = x_fea.reshape(B, T, DIN)
    adj16 = adj.astype(jnp.bfloat16)
    cand3 = candidate.astype(jnp.int32).reshape(B, NJ, 1)
    mask3 = mask_operation.astype(f32).reshape(B, NJ, 1)
    hgm3 = h_g_m_pooled.reshape(B, 1, H)
    pool_row = jnp.full((1, T), jnp.bfloat16(1.0 / T).astype(f32), f32)

    aw1a, aw1b, aw1c = a_w1[:H], a_w1[H:2 * H], a_w1[2 * H:]
    r2 = lambda v: v.reshape(1, -1)

    def full(w):
        nd = w.ndim
        return pl.BlockSpec(w.shape, lambda b, _n=nd: (0,) * _n)

    weights = (gin0_w1, r2(gin0_b1), gin0_w2, r2(gin0_b2),
               gin1_w1, r2(gin1_b1), gin1_w2, r2(gin1_b2),
               gin2_w1, r2(gin2_b1), gin2_w2, r2(gin2_b2),
               aw1a, aw1b, aw1c, r2(a_b1),
               a_w2, r2(a_b2), a_w3, r2(a_b3),
               c_w1, r2(c_b1), c_w2, r2(c_b2), c_w3, r2(c_b3))

    in_specs = [
        pl.BlockSpec((1, T, DIN), lambda b: (b, 0, 0)),
        pl.BlockSpec((1, T, T), lambda b: (b, 0, 0)),
        pl.BlockSpec((1, NJ, 1), lambda b: (b, 0, 0)),
        pl.BlockSpec((1, NJ, 1), lambda b: (b, 0, 0)),
        pl.BlockSpec((1, 1, H), lambda b: (b, 0, 0)),
        full(pool_row),
    ] + [full(w) for w in weights]

    out_shapes = (
        jax.ShapeDtypeStruct((B, 1, 1), jnp.int32),
        jax.ShapeDtypeStruct((B, 1, 1), jnp.int32),
        jax.ShapeDtypeStruct((B, 1, 1), f32),
        jax.ShapeDtypeStruct((B, NJ, 1), f32),
        jax.ShapeDtypeStruct((B, 1, H), f32),
        jax.ShapeDtypeStruct((B, 1, 4), f32),
    )
    out_specs = (
        pl.BlockSpec((1, 1, 1), lambda b: (b, 0, 0)),
        pl.BlockSpec((1, 1, 1), lambda b: (b, 0, 0)),
        pl.BlockSpec((1, 1, 1), lambda b: (b, 0, 0)),
        pl.BlockSpec((1, NJ, 1), lambda b: (b, 0, 0)),
        pl.BlockSpec((1, 1, H), lambda b: (b, 0, 0)),
        pl.BlockSpec((1, 1, 4), lambda b: (b, 0, 0)),
    )

    ti, ai, la, pr, hg, jv = pl.pallas_call(
        _fused_step,
        grid=(B,),
        in_specs=in_specs,
        out_specs=out_specs,
        out_shape=out_shapes,
    )(x3, adj16, cand3, mask3, hgm3, pool_row, *weights)

    return (ti.reshape(B), ai.reshape(B), la.reshape(B),
            pr.reshape(B, NJ), hg.reshape(B, H), jv.reshape(B, 4))


# speedup vs baseline: 1.2967x; 1.2426x over previous
import jax
import jax.numpy as jnp
from jax.experimental import pallas as pl

_HI = jax.lax.Precision.HIGHEST


def _bdot(a, b):
    return jnp.dot(a.astype(jnp.bfloat16), b.astype(jnp.bfloat16),
                   preferred_element_type=jnp.float32)


def _fused_step(
    x_ref, adj_ref, cand_ref, mask_ref, hgm_ref, pool_ref,
    g0w1_ref, g0b1_ref, g0w2_ref, g0b2_ref,
    g1w1_ref, g1b1_ref, g1w2_ref, g1b2_ref,
    g2w1_ref, g2b1_ref, g2w2_ref, g2b2_ref,
    aw1a_ref, aw1b_ref, aw1c_ref, ab1_ref,
    aw2_ref, ab2_ref, aw3_ref, ab3_ref,
    cw1_ref, cb1_ref, cw2_ref, cb2_ref, cw3_ref, cb3_ref,
    ti_ref, ai_ref, la_ref, pr_ref, hg_ref, jv_ref,
):
    f32 = jnp.float32
    a16 = adj_ref[0].astype(jnp.bfloat16)
    h = x_ref[0]
    T = a16.shape[0]

    gin = ((g0w1_ref, g0b1_ref, g0w2_ref, g0b2_ref),
           (g1w1_ref, g1b1_ref, g1w2_ref, g1b2_ref),
           (g2w1_ref, g2b1_ref, g2w2_ref, g2b2_ref))
    for w1_ref, b1_ref, w2_ref, b2_ref in gin:
        neigh = jnp.dot(a16, h.astype(jnp.bfloat16),
                        preferred_element_type=f32)
        pooled = neigh + h
        m = jnp.maximum(_bdot(pooled, w1_ref[...]) + b1_ref[...], 0.0)
        m = _bdot(m, w2_ref[...]) + b2_ref[...]
        h = jnp.maximum(m, 0.0)

    h16f = h.astype(jnp.bfloat16).astype(f32)
    hg = jnp.dot(pool_ref[...], h16f, preferred_element_type=f32,
                 precision=_HI)

    cand = cand_ref[0]
    nj = cand.shape[0]
    iota_t = jax.lax.broadcasted_iota(jnp.int32, (nj, T), 1)
    onehot = (iota_t == cand).astype(f32)
    cf = jnp.dot(onehot, h, preferred_element_type=f32, precision=_HI)

    hgm = hgm_ref[0]
    row = _bdot(hg, aw1b_ref[...]) + _bdot(hgm, aw1c_ref[...]) + ab1_ref[...]
    t1 = jnp.tanh(_bdot(cf, aw1a_ref[...]) + row)
    t2 = jnp.tanh(_bdot(t1, aw2_ref[...]) + ab2_ref[...])
    sc = _bdot(t2, aw3_ref[...]) + ab3_ref[...]
    sc = sc - mask_ref[0] * 1e30

    smax = jnp.max(sc, axis=0, keepdims=True)
    e = jnp.exp(sc - smax)
    esum = jnp.sum(e, axis=0, keepdims=True)
    prob = e / esum

    iota_nj = jax.lax.broadcasted_iota(jnp.int32, (nj, 1), 0)
    pmax = jnp.max(prob, axis=0, keepdims=True)
    am = jnp.min(jnp.where(prob == pmax, iota_nj, nj), axis=0, keepdims=True)
    task = jnp.sum(jnp.where(iota_nj == am, cand, 0), axis=0, keepdims=True)
    la = jnp.log(pmax + 1e-10)

    v1 = jnp.tanh(_bdot(hg, cw1_ref[...]) + cb1_ref[...])
    v2 = jnp.tanh(_bdot(v1, cw2_ref[...]) + cb2_ref[...])
    jv = _bdot(v2, cw3_ref[...]) + cb3_ref[...]

    ti_ref[...] = task.reshape(1, 1, 1)
    ai_ref[...] = am.reshape(1, 1, 1)
    la_ref[...] = la.reshape(1, 1, 1)
    pr_ref[...] = prob.reshape(1, nj, 1)
    hg_ref[...] = hg.reshape(1, 1, -1)
    jv_ref[...] = jv.reshape(1, 1, -1)


def kernel(x_fea, graph_pool_avg, padded_nei, adj, candidate, h_g_m_pooled,
           mask_operation,
           gin0_w1, gin0_b1, gin0_w2, gin0_b2,
           gin1_w1, gin1_b1, gin1_w2, gin1_b2,
           gin2_w1, gin2_b1, gin2_w2, gin2_b2,
           a_w1, a_b1, a_w2, a_b2, a_w3, a_b3,
           c_w1, c_b1, c_w2, c_b2, c_w3, c_b3):
    B, T, _ = adj.shape
    NJ = candidate.shape[1]
    DIN = x_fea.shape[1]
    H = gin0_w2.shape[0]
    f32 = jnp.float32

    x3 = x_fea.reshape(B, T, DIN)
    cand3 = candidate.astype(jnp.int32).reshape(B, NJ, 1)
    mask3 = mask_operation.astype(f32).reshape(B, NJ, 1)
    hgm3 = h_g_m_pooled.reshape(B, 1, H)
    pool_row = jnp.full((1, T), jnp.bfloat16(1.0 / T).astype(f32), f32)

    aw1a, aw1b, aw1c = a_w1[:H], a_w1[H:2 * H], a_w1[2 * H:]
    r2 = lambda v: v.reshape(1, -1)

    def full(w):
        nd = w.ndim
        return pl.BlockSpec(w.shape, lambda b, _n=nd: (0,) * _n)

    weights = (gin0_w1, r2(gin0_b1), gin0_w2, r2(gin0_b2),
               gin1_w1, r2(gin1_b1), gin1_w2, r2(gin1_b2),
               gin2_w1, r2(gin2_b1), gin2_w2, r2(gin2_b2),
               aw1a, aw1b, aw1c, r2(a_b1),
               a_w2, r2(a_b2), a_w3, r2(a_b3),
               c_w1, r2(c_b1), c_w2, r2(c_b2), c_w3, r2(c_b3))

    in_specs = [
        pl.BlockSpec((1, T, DIN), lambda b: (b, 0, 0)),
        pl.BlockSpec((1, T, T), lambda b: (b, 0, 0)),
        pl.BlockSpec((1, NJ, 1), lambda b: (b, 0, 0)),
        pl.BlockSpec((1, NJ, 1), lambda b: (b, 0, 0)),
        pl.BlockSpec((1, 1, H), lambda b: (b, 0, 0)),
        full(pool_row),
    ] + [full(w) for w in weights]

    out_shapes = (
        jax.ShapeDtypeStruct((B, 1, 1), jnp.int32),
        jax.ShapeDtypeStruct((B, 1, 1), jnp.int32),
        jax.ShapeDtypeStruct((B, 1, 1), f32),
        jax.ShapeDtypeStruct((B, NJ, 1), f32),
        jax.ShapeDtypeStruct((B, 1, H), f32),
        jax.ShapeDtypeStruct((B, 1, 4), f32),
    )
    out_specs = (
        pl.BlockSpec((1, 1, 1), lambda b: (b, 0, 0)),
        pl.BlockSpec((1, 1, 1), lambda b: (b, 0, 0)),
        pl.BlockSpec((1, 1, 1), lambda b: (b, 0, 0)),
        pl.BlockSpec((1, NJ, 1), lambda b: (b, 0, 0)),
        pl.BlockSpec((1, 1, H), lambda b: (b, 0, 0)),
        pl.BlockSpec((1, 1, 4), lambda b: (b, 0, 0)),
    )

    ti, ai, la, pr, hg, jv = pl.pallas_call(
        _fused_step,
        grid=(B,),
        in_specs=in_specs,
        out_specs=out_specs,
        out_shape=out_shapes,
    )(x3, adj, cand3, mask3, hgm3, pool_row, *weights)

    return (ti.reshape(B), ai.reshape(B), la.reshape(B),
            pr.reshape(B, NJ), hg.reshape(B, H), jv.reshape(B, 4))


# speedup vs baseline: 1.4219x; 1.0966x over previous
import jax
import jax.numpy as jnp
from jax.experimental import pallas as pl


def _fused_step(
    x_ref, adj_ref, cand_ref, mask_ref, hgm_ref, pool_ref,
    g0w1_ref, g0b1_ref, g0w2_ref, g0b2_ref,
    g1w1_ref, g1b1_ref, g1w2_ref, g1b2_ref,
    g2w1_ref, g2b1_ref, g2w2_ref, g2b2_ref,
    aw1a_ref, aw1b_ref, aw1c_ref, ab1_ref,
    aw2_ref, ab2_ref, aw3_ref, ab3_ref,
    cw1_ref, cb1_ref, cw2_ref, cb2_ref, cw3_ref, cb3_ref,
    ti_ref, ai_ref, la_ref, pr_ref, hg_ref, jv_ref,
):
    f32 = jnp.float32
    a = adj_ref[0]
    h = x_ref[0]
    T = a.shape[0]

    gin = ((g0w1_ref, g0b1_ref, g0w2_ref, g0b2_ref),
           (g1w1_ref, g1b1_ref, g1w2_ref, g1b2_ref),
           (g2w1_ref, g2b1_ref, g2w2_ref, g2b2_ref))
    for w1_ref, b1_ref, w2_ref, b2_ref in gin:
        neigh = jnp.dot(a, h, preferred_element_type=f32)
        pooled = neigh + h
        m = jnp.maximum(jnp.dot(pooled, w1_ref[...],
                                preferred_element_type=f32) + b1_ref[...], 0.0)
        m = jnp.dot(m, w2_ref[...], preferred_element_type=f32) + b2_ref[...]
        h = jnp.maximum(m, 0.0)

    hg = jnp.dot(pool_ref[...], h, preferred_element_type=f32)

    cand = cand_ref[0]
    nj = cand.shape[0]
    iota_t = jax.lax.broadcasted_iota(jnp.int32, (nj, T), 1)
    onehot = (iota_t == cand).astype(f32)
    cf = jnp.dot(onehot, h, preferred_element_type=f32)

    hgm = hgm_ref[0]
    row = (jnp.dot(hg, aw1b_ref[...], preferred_element_type=f32)
           + jnp.dot(hgm, aw1c_ref[...], preferred_element_type=f32)
           + ab1_ref[...])
    t1 = jnp.tanh(jnp.dot(cf, aw1a_ref[...], preferred_element_type=f32) + row)
    t2 = jnp.tanh(jnp.dot(t1, aw2_ref[...], preferred_element_type=f32) + ab2_ref[...])
    sc = jnp.dot(t2, aw3_ref[...], preferred_element_type=f32) + ab3_ref[...]
    sc = sc - mask_ref[0] * 1e30

    smax = jnp.max(sc, axis=0, keepdims=True)
    e = jnp.exp(sc - smax)
    esum = jnp.sum(e, axis=0, keepdims=True)
    prob = e / esum

    iota_nj = jax.lax.broadcasted_iota(jnp.int32, (nj, 1), 0)
    pmax = jnp.max(prob, axis=0, keepdims=True)
    am = jnp.min(jnp.where(prob == pmax, iota_nj, nj), axis=0, keepdims=True)
    task = jnp.sum(jnp.where(iota_nj == am, cand, 0), axis=0, keepdims=True)
    la = jnp.log(pmax + 1e-10)

    v1 = jnp.tanh(jnp.dot(hg, cw1_ref[...], preferred_element_type=f32) + cb1_ref[...])
    v2 = jnp.tanh(jnp.dot(v1, cw2_ref[...], preferred_element_type=f32) + cb2_ref[...])
    jv = jnp.dot(v2, cw3_ref[...], preferred_element_type=f32) + cb3_ref[...]

    ti_ref[...] = task.reshape(1, 1, 1)
    ai_ref[...] = am.reshape(1, 1, 1)
    la_ref[...] = la.reshape(1, 1, 1)
    pr_ref[...] = prob.reshape(1, nj, 1)
    hg_ref[...] = hg.reshape(1, 1, -1)
    jv_ref[...] = jv.reshape(1, 1, -1)


def kernel(x_fea, graph_pool_avg, padded_nei, adj, candidate, h_g_m_pooled,
           mask_operation,
           gin0_w1, gin0_b1, gin0_w2, gin0_b2,
           gin1_w1, gin1_b1, gin1_w2, gin1_b2,
           gin2_w1, gin2_b1, gin2_w2, gin2_b2,
           a_w1, a_b1, a_w2, a_b2, a_w3, a_b3,
           c_w1, c_b1, c_w2, c_b2, c_w3, c_b3):
    B, T, _ = adj.shape
    NJ = candidate.shape[1]
    DIN = x_fea.shape[1]
    H = gin0_w2.shape[0]
    f32 = jnp.float32

    x3 = x_fea.reshape(B, T, DIN)
    cand3 = candidate.astype(jnp.int32).reshape(B, NJ, 1)
    mask3 = mask_operation.astype(f32).reshape(B, NJ, 1)
    hgm3 = h_g_m_pooled.reshape(B, 1, H)
    pool_row = jnp.full((1, T), 1.0 / T, f32)

    aw1a, aw1b, aw1c = a_w1[:H], a_w1[H:2 * H], a_w1[2 * H:]
    r2 = lambda v: v.reshape(1, -1)

    def full(w):
        nd = w.ndim
        return pl.BlockSpec(w.shape, lambda b, _n=nd: (0,) * _n)

    weights = (gin0_w1, r2(gin0_b1), gin0_w2, r2(gin0_b2),
               gin1_w1, r2(gin1_b1), gin1_w2, r2(gin1_b2),
               gin2_w1, r2(gin2_b1), gin2_w2, r2(gin2_b2),
               aw1a, aw1b, aw1c, r2(a_b1),
               a_w2, r2(a_b2), a_w3, r2(a_b3),
               c_w1, r2(c_b1), c_w2, r2(c_b2), c_w3, r2(c_b3))

    in_specs = [
        pl.BlockSpec((1, T, DIN), lambda b: (b, 0, 0)),
        pl.BlockSpec((1, T, T), lambda b: (b, 0, 0)),
        pl.BlockSpec((1, NJ, 1), lambda b: (b, 0, 0)),
        pl.BlockSpec((1, NJ, 1), lambda b: (b, 0, 0)),
        pl.BlockSpec((1, 1, H), lambda b: (b, 0, 0)),
        full(pool_row),
    ] + [full(w) for w in weights]

    out_shapes = (
        jax.ShapeDtypeStruct((B, 1, 1), jnp.int32),
        jax.ShapeDtypeStruct((B, 1, 1), jnp.int32),
        jax.ShapeDtypeStruct((B, 1, 1), f32),
        jax.ShapeDtypeStruct((B, NJ, 1), f32),
        jax.ShapeDtypeStruct((B, 1, H), f32),
        jax.ShapeDtypeStruct((B, 1, 4), f32),
    )
    out_specs = (
        pl.BlockSpec((1, 1, 1), lambda b: (b, 0, 0)),
        pl.BlockSpec((1, 1, 1), lambda b: (b, 0, 0)),
        pl.BlockSpec((1, 1, 1), lambda b: (b, 0, 0)),
        pl.BlockSpec((1, NJ, 1), lambda b: (b, 0, 0)),
        pl.BlockSpec((1, 1, H), lambda b: (b, 0, 0)),
        pl.BlockSpec((1, 1, 4), lambda b: (b, 0, 0)),
    )

    ti, ai, la, pr, hg, jv = pl.pallas_call(
        _fused_step,
        grid=(B,),
        in_specs=in_specs,
        out_specs=out_specs,
        out_shape=out_shapes,
    )(x3, adj, cand3, mask3, hgm3, pool_row, *weights)

    return (ti.reshape(B), ai.reshape(B), la.reshape(B),
            pr.reshape(B, NJ), hg.reshape(B, H), jv.reshape(B, 4))
